# baseline (speedup 1.0000x reference)
import functools

import jax
import jax.numpy as jnp
from jax import lax
from jax.experimental import pallas as pl
from jax.experimental.pallas import tpu as pltpu

_K = 64
_N = 32768
_ROWS = 128
_RBLK = 16
_MASK31 = 0x7FFFFFFF
_BIAS30 = 1 << 30


def _sortable(z):
    b = lax.bitcast_convert_type(z, jnp.int32)
    return b ^ (lax.shift_right_arithmetic(b, 31) & _MASK31)


def _topk_block(z_ref, o_ref, s_ref):
    s_ref[...] = lax.shift_right_arithmetic(_sortable(z_ref[...]), 1)

    def count_ge(c_s):
        sv = s_ref[...]
        acc = lax.shift_right_arithmetic(sv[:, 0:1024] - c_s, 31)
        for a in range(1, _N // 1024):
            acc = acc + lax.shift_right_arithmetic(
                sv[:, a * 1024:(a + 1) * 1024] - c_s, 31)
        return _N + jnp.sum(acc, axis=1, keepdims=True)

    zeros = jnp.zeros((_RBLK, 1), jnp.int32)

    gm = jnp.max(s_ref[...].reshape(_RBLK, _N // 512, 512), axis=1)
    gm = jnp.maximum(jnp.maximum(gm[:, 0:128], gm[:, 128:256]),
                     jnp.maximum(gm[:, 256:384], gm[:, 384:512]))
    l_u = jnp.min(gm, axis=1, keepdims=True) + _BIAS30
    u_u = jnp.max(gm, axis=1, keepdims=True) + _BIAS30
    x = l_u ^ u_u
    e = lax.shift_right_arithmetic(
        lax.bitcast_convert_type(x.astype(jnp.float32), jnp.int32),
        23) - 127
    low_mask = jnp.where(
        x == 0, 0,
        lax.shift_left(jnp.int32(2), jnp.maximum(e, 0)) - 1)
    v_init = u_u & ~low_mask
    p_start = jnp.max(e)

    def search_cond(state):
        p, _, done, _ = state
        return jnp.logical_and(p >= 0, jnp.logical_not(jnp.all(done == 1)))

    def search_body(state):
        p, v_u, done, thr = state
        bit = lax.shift_left(jnp.int32(1), p)
        c_u = v_u | bit
        cnt = count_ge(c_u - _BIAS30)
        pinned = (cnt == _K) & (done == 0)
        thr = jnp.where(pinned, c_u, thr)
        done = done | pinned.astype(jnp.int32)
        v_u = jnp.where(cnt >= _K, c_u, v_u)
        return p - 1, v_u, done, thr

    def fori_body(i, state):
        return search_body(state)

    _, v_u, done, thr = lax.fori_loop(
        0, 31, fori_body, (jnp.int32(30), zeros, zeros, zeros))

    all_pinned = jnp.all(done == 1)

    @pl.when(all_pinned)
    def _fast():
        o_ref[...] = jnp.where(s_ref[...] >= thr - _BIAS30,
                               z_ref[...], 0.0)

    @pl.when(jnp.logical_not(all_pinned))
    def _general():
        sf = _sortable(z_ref[...])
        v31_s = v_u - _BIAS30
        cnt_gt31 = jnp.sum((s_ref[...] > v31_s).astype(jnp.int32),
                           axis=1, keepdims=True)
        m31 = _K - cnt_gt31
        cnt_odd = jnp.sum((sf == 2 * v31_s + 1).astype(jnp.int32),
                          axis=1, keepdims=True)
        v_full = jnp.where(cnt_odd >= m31, 2 * v31_s + 1, 2 * v31_s)
        v_full = jnp.where(done == 1, 2 * (thr - _BIAS30), v_full)

        cnt_gt = jnp.sum((sf > v_full).astype(jnp.int32), axis=1,
                         keepdims=True)
        m = _K - cnt_gt
        cnt_eq = jnp.sum((sf == v_full).astype(jnp.int32), axis=1,
                         keepdims=True)

        def tie_branch(_):
            def idx_step(i, c):
                t = c | lax.shift_left(jnp.int32(1), jnp.int32(16) - i)
                sv = _sortable(z_ref[...])
                idx = lax.broadcasted_iota(jnp.int32, (_RBLK, _N), 1)
                g = jnp.sum(((sv == v_full) & (idx < t)).astype(jnp.int32),
                            axis=1, keepdims=True)
                return jnp.where(g <= m, t, c)

            return lax.fori_loop(0, 17, idx_step, zeros)

        c_idx = lax.cond(jnp.all(cnt_eq <= m),
                         lambda _: jnp.full((_RBLK, 1), _N, jnp.int32),
                         tie_branch, None)

        idx = lax.broadcasted_iota(jnp.int32, (_RBLK, _N), 1)
        keep = (sf > v_full) | ((sf == v_full) & (idx < c_idx))
        o_ref[...] = jnp.where(keep, z_ref[...], 0.0)


@jax.jit
def kernel(z):
    grid = (_ROWS // _RBLK,)
    return pl.pallas_call(
        _topk_block,
        grid=grid,
        in_specs=[pl.BlockSpec((_RBLK, _N), lambda i: (i, 0))],
        out_specs=pl.BlockSpec((_RBLK, _N), lambda i: (i, 0)),
        out_shape=jax.ShapeDtypeStruct((_ROWS, _N), jnp.float32),
        scratch_shapes=[pltpu.VMEM((_RBLK, _N), jnp.int32)],
        compiler_params=pltpu.CompilerParams(
            dimension_semantics=("parallel",)),
    )(z)


# speedup vs baseline: 1.2001x; 1.2001x over previous
import functools

import jax
import jax.numpy as jnp
from jax import lax
from jax.experimental import pallas as pl
from jax.experimental.pallas import tpu as pltpu

_K = 64
_N = 32768
_ROWS = 128
_RBLK = 16
_MASK31 = 0x7FFFFFFF
_BIAS30 = 1 << 30


def _sortable(z):
    b = lax.bitcast_convert_type(z, jnp.int32)
    return b ^ (lax.shift_right_arithmetic(b, 31) & _MASK31)


def _topk_block(z_ref, o_ref, s_ref):
    s_ref[...] = lax.shift_right_arithmetic(_sortable(z_ref[...]), 1)

    def count_ge(c_s):
        sv = s_ref[...]
        acc = lax.shift_right_arithmetic(sv[:, 0:1024] - c_s, 31)
        for a in range(1, _N // 1024):
            acc = acc + lax.shift_right_arithmetic(
                sv[:, a * 1024:(a + 1) * 1024] - c_s, 31)
        return _N + jnp.sum(acc, axis=1, keepdims=True)

    zeros = jnp.zeros((_RBLK, 1), jnp.int32)

    sv = s_ref[...]
    gm = sv[:, 0:1024]
    for a in range(1, _N // 1024):
        gm = jnp.maximum(gm, sv[:, a * 1024:(a + 1) * 1024])
    gm = jnp.maximum(gm[:, 0:512], gm[:, 512:1024])
    gm = jnp.maximum(gm[:, 0:256], gm[:, 256:512])
    gm = jnp.maximum(gm[:, 0:128], gm[:, 128:256])
    l_u = jnp.min(gm, axis=1, keepdims=True) + _BIAS30
    u_u = jnp.max(gm, axis=1, keepdims=True) + _BIAS30
    x = l_u ^ u_u
    e = lax.shift_right_arithmetic(
        lax.bitcast_convert_type(x.astype(jnp.float32), jnp.int32),
        23) - 127
    low_mask = jnp.where(
        x == 0, 0,
        lax.shift_left(jnp.int32(2), jnp.maximum(e, 0)) - 1)
    v_init = u_u & ~low_mask
    p_start = jnp.max(e)

    def search_cond(state):
        p, _, done, _ = state
        return jnp.logical_and(p >= 0, jnp.logical_not(jnp.all(done == 1)))

    def search_body(state):
        p, v_u, done, thr = state
        bit = lax.shift_left(jnp.int32(1), p)
        c_u = v_u | bit
        cnt = count_ge(c_u - _BIAS30)
        pinned = (cnt == _K) & (done == 0)
        thr = jnp.where(pinned, c_u, thr)
        done = done | pinned.astype(jnp.int32)
        v_u = jnp.where(cnt >= _K, c_u, v_u)
        return p - 1, v_u, done, thr

    _, v_u, done, thr = lax.while_loop(
        search_cond, search_body, (p_start, v_init, zeros, zeros))

    all_pinned = jnp.all(done == 1)

    @pl.when(all_pinned)
    def _fast():
        o_ref[...] = jnp.where(s_ref[...] >= thr - _BIAS30,
                               z_ref[...], 0.0)

    @pl.when(jnp.logical_not(all_pinned))
    def _general():
        sf = _sortable(z_ref[...])
        v31_s = v_u - _BIAS30
        cnt_gt31 = jnp.sum((s_ref[...] > v31_s).astype(jnp.int32),
                           axis=1, keepdims=True)
        m31 = _K - cnt_gt31
        cnt_odd = jnp.sum((sf == 2 * v31_s + 1).astype(jnp.int32),
                          axis=1, keepdims=True)
        v_full = jnp.where(cnt_odd >= m31, 2 * v31_s + 1, 2 * v31_s)
        v_full = jnp.where(done == 1, 2 * (thr - _BIAS30), v_full)

        cnt_gt = jnp.sum((sf > v_full).astype(jnp.int32), axis=1,
                         keepdims=True)
        m = _K - cnt_gt
        cnt_eq = jnp.sum((sf == v_full).astype(jnp.int32), axis=1,
                         keepdims=True)

        def tie_branch(_):
            def idx_step(i, c):
                t = c | lax.shift_left(jnp.int32(1), jnp.int32(16) - i)
                sv = _sortable(z_ref[...])
                idx = lax.broadcasted_iota(jnp.int32, (_RBLK, _N), 1)
                g = jnp.sum(((sv == v_full) & (idx < t)).astype(jnp.int32),
                            axis=1, keepdims=True)
                return jnp.where(g <= m, t, c)

            return lax.fori_loop(0, 17, idx_step, zeros)

        c_idx = lax.cond(jnp.all(cnt_eq <= m),
                         lambda _: jnp.full((_RBLK, 1), _N, jnp.int32),
                         tie_branch, None)

        idx = lax.broadcasted_iota(jnp.int32, (_RBLK, _N), 1)
        keep = (sf > v_full) | ((sf == v_full) & (idx < c_idx))
        o_ref[...] = jnp.where(keep, z_ref[...], 0.0)


@jax.jit
def kernel(z):
    grid = (_ROWS // _RBLK,)
    return pl.pallas_call(
        _topk_block,
        grid=grid,
        in_specs=[pl.BlockSpec((_RBLK, _N), lambda i: (i, 0))],
        out_specs=pl.BlockSpec((_RBLK, _N), lambda i: (i, 0)),
        out_shape=jax.ShapeDtypeStruct((_ROWS, _N), jnp.float32),
        scratch_shapes=[pltpu.VMEM((_RBLK, _N), jnp.int32)],
        compiler_params=pltpu.CompilerParams(
            dimension_semantics=("parallel",)),
    )(z)


# speedup vs baseline: 1.2040x; 1.0033x over previous
import functools

import jax
import jax.numpy as jnp
from jax import lax
from jax.experimental import pallas as pl
from jax.experimental.pallas import tpu as pltpu

_K = 64
_N = 32768
_ROWS = 128
_RBLK = 16
_MASK31 = 0x7FFFFFFF
_BIAS30 = 1 << 30


def _sortable(z):
    b = lax.bitcast_convert_type(z, jnp.int32)
    return b ^ (lax.shift_right_arithmetic(b, 31) & _MASK31)


def _topk_block(z_ref, o_ref, s_ref):
    s_ref[...] = lax.shift_right_arithmetic(_sortable(z_ref[...]), 1)

    def count_ge(c_s):
        sv = s_ref[...]
        acc = lax.shift_right_arithmetic(sv[:, 0:1024] - c_s, 31)
        for a in range(1, _N // 1024):
            acc = acc + lax.shift_right_arithmetic(
                sv[:, a * 1024:(a + 1) * 1024] - c_s, 31)
        return _N + jnp.sum(acc, axis=1, keepdims=True)

    zeros = jnp.zeros((_RBLK, 1), jnp.int32)

    sv = s_ref[...]
    gm = sv[:, 0:1024]
    for a in range(1, _N // 1024):
        gm = jnp.maximum(gm, sv[:, a * 1024:(a + 1) * 1024])
    gm = jnp.maximum(gm[:, 0:512], gm[:, 512:1024])
    gm = jnp.maximum(gm[:, 0:256], gm[:, 256:512])
    gm = jnp.maximum(gm[:, 0:128], gm[:, 128:256])
    l_u = jnp.min(gm, axis=1, keepdims=True) + _BIAS30
    u_u = jnp.max(gm, axis=1, keepdims=True) + _BIAS30
    x = l_u ^ u_u
    e = lax.shift_right_arithmetic(
        lax.bitcast_convert_type(x.astype(jnp.float32), jnp.int32),
        23) - 127
    low_mask = jnp.where(
        x == 0, 0,
        lax.shift_left(jnp.int32(2), jnp.maximum(e, 0)) - 1)
    v_init = u_u & ~low_mask
    x_all = (jnp.min(gm) + _BIAS30) ^ (jnp.max(gm) + _BIAS30)
    p_start = lax.shift_right_arithmetic(
        lax.bitcast_convert_type(x_all.astype(jnp.float32), jnp.int32),
        23) - 127

    def search_cond(state):
        p, _, done, _ = state
        return jnp.logical_and(p >= 0, jnp.logical_not(jnp.all(done == 1)))

    def search_body(state):
        p, v_u, done, thr = state
        bit = lax.shift_left(jnp.int32(1), p)
        c_u = v_u | bit
        cnt = count_ge(c_u - _BIAS30)
        pinned = (cnt == _K) & (done == 0)
        thr = jnp.where(pinned, c_u, thr)
        done = done | pinned.astype(jnp.int32)
        v_u = jnp.where(cnt >= _K, c_u, v_u)
        return p - 1, v_u, done, thr

    _, v_u, done, thr = lax.while_loop(
        search_cond, search_body, (p_start, v_init, zeros, zeros))

    all_pinned = jnp.all(done == 1)

    @pl.when(all_pinned)
    def _fast():
        o_ref[...] = jnp.where(s_ref[...] >= thr - _BIAS30,
                               z_ref[...], 0.0)

    @pl.when(jnp.logical_not(all_pinned))
    def _general():
        sf = _sortable(z_ref[...])
        v31_s = v_u - _BIAS30
        cnt_gt31 = jnp.sum((s_ref[...] > v31_s).astype(jnp.int32),
                           axis=1, keepdims=True)
        m31 = _K - cnt_gt31
        cnt_odd = jnp.sum((sf == 2 * v31_s + 1).astype(jnp.int32),
                          axis=1, keepdims=True)
        v_full = jnp.where(cnt_odd >= m31, 2 * v31_s + 1, 2 * v31_s)
        v_full = jnp.where(done == 1, 2 * (thr - _BIAS30), v_full)

        cnt_gt = jnp.sum((sf > v_full).astype(jnp.int32), axis=1,
                         keepdims=True)
        m = _K - cnt_gt
        cnt_eq = jnp.sum((sf == v_full).astype(jnp.int32), axis=1,
                         keepdims=True)

        def tie_branch(_):
            def idx_step(i, c):
                t = c | lax.shift_left(jnp.int32(1), jnp.int32(16) - i)
                sv = _sortable(z_ref[...])
                idx = lax.broadcasted_iota(jnp.int32, (_RBLK, _N), 1)
                g = jnp.sum(((sv == v_full) & (idx < t)).astype(jnp.int32),
                            axis=1, keepdims=True)
                return jnp.where(g <= m, t, c)

            return lax.fori_loop(0, 17, idx_step, zeros)

        c_idx = lax.cond(jnp.all(cnt_eq <= m),
                         lambda _: jnp.full((_RBLK, 1), _N, jnp.int32),
                         tie_branch, None)

        idx = lax.broadcasted_iota(jnp.int32, (_RBLK, _N), 1)
        keep = (sf > v_full) | ((sf == v_full) & (idx < c_idx))
        o_ref[...] = jnp.where(keep, z_ref[...], 0.0)


@jax.jit
def kernel(z):
    grid = (_ROWS // _RBLK,)
    return pl.pallas_call(
        _topk_block,
        grid=grid,
        in_specs=[pl.BlockSpec((_RBLK, _N), lambda i: (i, 0))],
        out_specs=pl.BlockSpec((_RBLK, _N), lambda i: (i, 0)),
        out_shape=jax.ShapeDtypeStruct((_ROWS, _N), jnp.float32),
        scratch_shapes=[pltpu.VMEM((_RBLK, _N), jnp.int32)],
        compiler_params=pltpu.CompilerParams(
            dimension_semantics=("parallel",)),
    )(z)


# speedup vs baseline: 1.3571x; 1.1271x over previous
import functools

import jax
import jax.numpy as jnp
from jax import lax
from jax.experimental import pallas as pl
from jax.experimental.pallas import tpu as pltpu

_K = 64
_N = 32768
_ROWS = 128
_RBLK = 16
_MASK31 = 0x7FFFFFFF
_BIAS30 = 1 << 30


def _sortable(z):
    b = lax.bitcast_convert_type(z, jnp.int32)
    return b ^ (lax.shift_right_arithmetic(b, 31) & _MASK31)


def _topk_block(z_ref, o_ref, s_ref):
    s_ref[...] = lax.shift_right_arithmetic(_sortable(z_ref[...]), 1)

    def count_ge(c_s):
        sv = s_ref[...]
        acc = lax.shift_right_arithmetic(sv[:, 0:1024] - c_s, 31)
        for a in range(1, _N // 1024):
            acc = acc + lax.shift_right_arithmetic(
                sv[:, a * 1024:(a + 1) * 1024] - c_s, 31)
        return _N + jnp.sum(acc, axis=1, keepdims=True)

    zeros = jnp.zeros((_RBLK, 1), jnp.int32)

    sv = s_ref[...]
    gm = sv[:, 0:512]
    for a in range(1, _N // 512):
        gm = jnp.maximum(gm, sv[:, a * 512:(a + 1) * 512])
    def gm_step(i, l_b):
        bit = lax.shift_left(jnp.int32(1), jnp.int32(30) - i)
        c_u = l_b | bit
        d = lax.shift_right_arithmetic(gm - (c_u - _BIAS30), 31)
        cnt = 512 + jnp.sum(d, axis=1, keepdims=True)
        return jnp.where(cnt >= _K, c_u, l_b)

    l_u = lax.fori_loop(0, 20, gm_step, zeros)
    u_u = jnp.max(gm, axis=1, keepdims=True) + _BIAS30
    x = l_u ^ u_u
    e = lax.shift_right_arithmetic(
        lax.bitcast_convert_type(x.astype(jnp.float32), jnp.int32),
        23) - 127
    low_mask = jnp.where(
        x == 0, 0,
        lax.shift_left(jnp.int32(2), jnp.maximum(e, 0)) - 1)
    v_init = u_u & ~low_mask
    x_all = (jnp.min(jnp.broadcast_to(l_u, (_RBLK, 128)))
             ^ jnp.max(jnp.broadcast_to(u_u, (_RBLK, 128))))
    p_start = lax.shift_right_arithmetic(
        lax.bitcast_convert_type(x_all.astype(jnp.float32), jnp.int32),
        23) - 127

    def search_cond(state):
        p, _, done, _ = state
        return jnp.logical_and(p >= 0, jnp.logical_not(jnp.all(done == 1)))

    def search_body(state):
        p, v_u, done, thr = state
        bit = lax.shift_left(jnp.int32(1), p)
        c_u = v_u | bit
        cnt = count_ge(c_u - _BIAS30)
        pinned = (cnt == _K) & (done == 0)
        thr = jnp.where(pinned, c_u, thr)
        done = done | pinned.astype(jnp.int32)
        v_u = jnp.where(cnt >= _K, c_u, v_u)
        return p - 1, v_u, done, thr

    _, v_u, done, thr = lax.while_loop(
        search_cond, search_body, (p_start, v_init, zeros, zeros))

    all_pinned = jnp.all(done == 1)

    @pl.when(all_pinned)
    def _fast():
        o_ref[...] = jnp.where(s_ref[...] >= thr - _BIAS30,
                               z_ref[...], 0.0)

    @pl.when(jnp.logical_not(all_pinned))
    def _general():
        sf = _sortable(z_ref[...])
        v31_s = v_u - _BIAS30
        cnt_gt31 = jnp.sum((s_ref[...] > v31_s).astype(jnp.int32),
                           axis=1, keepdims=True)
        m31 = _K - cnt_gt31
        cnt_odd = jnp.sum((sf == 2 * v31_s + 1).astype(jnp.int32),
                          axis=1, keepdims=True)
        v_full = jnp.where(cnt_odd >= m31, 2 * v31_s + 1, 2 * v31_s)
        v_full = jnp.where(done == 1, 2 * (thr - _BIAS30), v_full)

        cnt_gt = jnp.sum((sf > v_full).astype(jnp.int32), axis=1,
                         keepdims=True)
        m = _K - cnt_gt
        cnt_eq = jnp.sum((sf == v_full).astype(jnp.int32), axis=1,
                         keepdims=True)

        def tie_branch(_):
            def idx_step(i, c):
                t = c | lax.shift_left(jnp.int32(1), jnp.int32(16) - i)
                sv = _sortable(z_ref[...])
                idx = lax.broadcasted_iota(jnp.int32, (_RBLK, _N), 1)
                g = jnp.sum(((sv == v_full) & (idx < t)).astype(jnp.int32),
                            axis=1, keepdims=True)
                return jnp.where(g <= m, t, c)

            return lax.fori_loop(0, 17, idx_step, zeros)

        c_idx = lax.cond(jnp.all(cnt_eq <= m),
                         lambda _: jnp.full((_RBLK, 1), _N, jnp.int32),
                         tie_branch, None)

        idx = lax.broadcasted_iota(jnp.int32, (_RBLK, _N), 1)
        keep = (sf > v_full) | ((sf == v_full) & (idx < c_idx))
        o_ref[...] = jnp.where(keep, z_ref[...], 0.0)


@jax.jit
def kernel(z):
    grid = (_ROWS // _RBLK,)
    return pl.pallas_call(
        _topk_block,
        grid=grid,
        in_specs=[pl.BlockSpec((_RBLK, _N), lambda i: (i, 0))],
        out_specs=pl.BlockSpec((_RBLK, _N), lambda i: (i, 0)),
        out_shape=jax.ShapeDtypeStruct((_ROWS, _N), jnp.float32),
        scratch_shapes=[pltpu.VMEM((_RBLK, _N), jnp.int32)],
        compiler_params=pltpu.CompilerParams(
            dimension_semantics=("parallel",)),
    )(z)


# speedup vs baseline: 1.3917x; 1.0255x over previous
import functools

import jax
import jax.numpy as jnp
from jax import lax
from jax.experimental import pallas as pl
from jax.experimental.pallas import tpu as pltpu

_K = 64
_N = 32768
_ROWS = 128
_RBLK = 16
_MASK31 = 0x7FFFFFFF
_BIAS30 = 1 << 30


def _sortable(z):
    b = lax.bitcast_convert_type(z, jnp.int32)
    return b ^ (lax.shift_right_arithmetic(b, 31) & _MASK31)


def _to_float(s32):
    b = s32 ^ (lax.shift_right_arithmetic(s32, 31) & _MASK31)
    return lax.bitcast_convert_type(b, jnp.float32)


def _key31_to_float(c_u):
    s32 = jnp.clip(lax.shift_left(c_u - _BIAS30, 1),
                   -2139095041, 2139095040)
    return _to_float(s32)


def _topk_block(z_ref, o_ref):
    zv = z_ref[...]

    def count_ge(c_u):
        c_f = _key31_to_float(c_u)
        zs = z_ref[...]
        acc = lax.shift_right_arithmetic(
            lax.bitcast_convert_type(zs[:, 0:1024] - c_f, jnp.int32), 31)
        for a in range(1, _N // 1024):
            acc = acc + lax.shift_right_arithmetic(
                lax.bitcast_convert_type(
                    zs[:, a * 1024:(a + 1) * 1024] - c_f, jnp.int32), 31)
        return _N + jnp.sum(acc, axis=1, keepdims=True)

    zeros = jnp.zeros((_RBLK, 1), jnp.int32)

    gmf = zv[:, 0:512]
    for a in range(1, _N // 512):
        gmf = jnp.maximum(gmf, zv[:, a * 512:(a + 1) * 512])
    gm = lax.shift_right_arithmetic(_sortable(gmf), 1)

    def gm_step(i, l_b):
        bit = lax.shift_left(jnp.int32(1), jnp.int32(30) - i)
        c_u = l_b | bit
        d = lax.shift_right_arithmetic(gm - (c_u - _BIAS30), 31)
        cnt = 512 + jnp.sum(d, axis=1, keepdims=True)
        return jnp.where(cnt >= _K, c_u, l_b)

    l_u = lax.fori_loop(0, 20, gm_step, zeros)
    u_u = jnp.max(gm, axis=1, keepdims=True) + _BIAS30
    x = l_u ^ u_u
    e = lax.shift_right_arithmetic(
        lax.bitcast_convert_type(x.astype(jnp.float32), jnp.int32),
        23) - 127
    low_mask = jnp.where(
        x == 0, 0,
        lax.shift_left(jnp.int32(2), jnp.maximum(e, 0)) - 1)
    v_init = u_u & ~low_mask
    x_all = (jnp.min(jnp.broadcast_to(l_u, (_RBLK, 128)))
             ^ jnp.max(jnp.broadcast_to(u_u, (_RBLK, 128))))
    p_start = lax.shift_right_arithmetic(
        lax.bitcast_convert_type(x_all.astype(jnp.float32), jnp.int32),
        23) - 127

    def search_cond(state):
        p, _, done, _ = state
        return jnp.logical_and(p >= 0, jnp.logical_not(jnp.all(done == 1)))

    def search_body(state):
        p, v_u, done, thr = state
        bit = lax.shift_left(jnp.int32(1), p)
        c_u = v_u | bit
        cnt = count_ge(c_u)
        pinned = (cnt == _K) & (done == 0)
        thr = jnp.where(pinned, c_u, thr)
        done = done | pinned.astype(jnp.int32)
        v_u = jnp.where(cnt >= _K, c_u, v_u)
        return p - 1, v_u, done, thr

    _, v_u, done, thr = lax.while_loop(
        search_cond, search_body, (p_start, v_init, zeros, zeros))

    all_pinned = jnp.all(done == 1)

    @pl.when(all_pinned)
    def _fast():
        thr_f = _key31_to_float(thr)
        zb = lax.bitcast_convert_type(z_ref[...], jnp.int32)
        neg = lax.shift_right_arithmetic(
            lax.bitcast_convert_type(z_ref[...] - thr_f, jnp.int32), 31)
        o_ref[...] = lax.bitcast_convert_type(zb & ~neg, jnp.float32)

    @pl.when(jnp.logical_not(all_pinned))
    def _general():
        sf = _sortable(z_ref[...])
        s31 = lax.shift_right_arithmetic(sf, 1)
        v31_s = v_u - _BIAS30
        cnt_gt31 = jnp.sum((s31 > v31_s).astype(jnp.int32),
                           axis=1, keepdims=True)
        m31 = _K - cnt_gt31
        cnt_odd = jnp.sum((sf == 2 * v31_s + 1).astype(jnp.int32),
                          axis=1, keepdims=True)
        v_full = jnp.where(cnt_odd >= m31, 2 * v31_s + 1, 2 * v31_s)
        v_full = jnp.where(done == 1, 2 * (thr - _BIAS30), v_full)

        cnt_gt = jnp.sum((sf > v_full).astype(jnp.int32), axis=1,
                         keepdims=True)
        m = _K - cnt_gt
        cnt_eq = jnp.sum((sf == v_full).astype(jnp.int32), axis=1,
                         keepdims=True)

        def tie_branch(_):
            def idx_step(i, c):
                t = c | lax.shift_left(jnp.int32(1), jnp.int32(16) - i)
                sv = _sortable(z_ref[...])
                idx = lax.broadcasted_iota(jnp.int32, (_RBLK, _N), 1)
                g = jnp.sum(((sv == v_full) & (idx < t)).astype(jnp.int32),
                            axis=1, keepdims=True)
                return jnp.where(g <= m, t, c)

            return lax.fori_loop(0, 17, idx_step, zeros)

        c_idx = lax.cond(jnp.all(cnt_eq <= m),
                         lambda _: jnp.full((_RBLK, 1), _N, jnp.int32),
                         tie_branch, None)

        idx = lax.broadcasted_iota(jnp.int32, (_RBLK, _N), 1)
        keep = (sf > v_full) | ((sf == v_full) & (idx < c_idx))
        o_ref[...] = jnp.where(keep, z_ref[...], 0.0)


@jax.jit
def kernel(z):
    grid = (_ROWS // _RBLK,)
    return pl.pallas_call(
        _topk_block,
        grid=grid,
        in_specs=[pl.BlockSpec((_RBLK, _N), lambda i: (i, 0))],
        out_specs=pl.BlockSpec((_RBLK, _N), lambda i: (i, 0)),
        out_shape=jax.ShapeDtypeStruct((_ROWS, _N), jnp.float32),
        compiler_params=pltpu.CompilerParams(
            dimension_semantics=("parallel",)),
    )(z)
